# Initial kernel scaffold; baseline (speedup 1.0000x reference)
import jax
import jax.numpy as jnp
from jax.experimental import pallas as pl


def kernel(input_ids, token_table, position_table):
    raise NotImplementedError("write your pallas kernel here")


# speedup vs baseline: 7.6462x; 7.6462x over previous
import functools

import jax
import jax.numpy as jnp
from jax import lax
from jax.experimental import pallas as pl
from jax.experimental.pallas import tpu as pltpu
from jax.experimental.pallas import tpu_sc as plsc

_VOCAB = 100000
_EMBED = 128
_SEQ = 200
_BATCH = 4096
_HALF = _SEQ // 2
_LANES = 16


def _make_kernel():
    info = plsc.get_sparse_core_info()
    nc, ns = info.num_cores, info.num_subcores
    nw = nc * ns
    rows = _BATCH * _SEQ
    per_w = rows // nw
    nseq_w = per_w // _SEQ

    mesh = plsc.VectorSubcoreMesh(core_axis_name="c", subcore_axis_name="s")

    @functools.partial(
        pl.kernel,
        mesh=mesh,
        out_type=jax.ShapeDtypeStruct((rows, _EMBED), jnp.float32),
        scratch_types=[
            pltpu.VMEM((_SEQ, _EMBED), jnp.float32),
            pltpu.VMEM((2, 2, _HALF), jnp.int32),
            pltpu.VMEM((2, _SEQ, _EMBED), jnp.float32),
            pltpu.SemaphoreType.DMA,
            pltpu.SemaphoreType.DMA,
            pltpu.SemaphoreType.DMA,
            pltpu.SemaphoreType.DMA,
            pltpu.SemaphoreType.DMA,
            pltpu.SemaphoreType.DMA,
        ],
    )
    def kern(ids_hbm, tok_hbm, pos_hbm, out_hbm, pos_v, idx_v, rows_v,
             si0, si1, sg0, sg1, ss0, ss1):
        sem_i = (si0, si1)
        sem_g = (sg0, sg1)
        sem_s = (ss0, ss1)
        wid = lax.axis_index("s") * nc + lax.axis_index("c")
        row0 = wid * per_w
        irow0 = wid * (per_w // _HALF)

        def idx_dma(g, b):
            return pltpu.make_async_copy(
                ids_hbm.at[pl.ds(irow0 + 2 * g, 2)], idx_v.at[b], sem_i[b])

        def gather_dma(b, h):
            return pltpu.make_async_copy(
                tok_hbm.at[idx_v.at[b, h]],
                rows_v.at[b, pl.ds(h * _HALF, _HALF)], sem_g[b])

        def scatter_dma(g, b):
            return pltpu.make_async_copy(
                rows_v.at[b], out_hbm.at[pl.ds(row0 + g * _SEQ, _SEQ)],
                sem_s[b])

        def gather_start(b):
            gather_dma(b, 0).start()
            gather_dma(b, 1).start()

        def gather_wait(b):
            gather_dma(b, 0).wait()
            gather_dma(b, 1).wait()

        def add_pos(b):
            def rbody(r, _):
                for c in range(_EMBED // _LANES):
                    sl = pl.ds(c * _LANES, _LANES)
                    plsc.addupdate(rows_v.at[b, r, sl], pos_v[r, sl])
                return _
            lax.fori_loop(0, _SEQ, rbody, 0, unroll=2)

        pltpu.sync_copy(pos_hbm.at[pl.ds(0, _SEQ)], pos_v)

        idx_dma(0, 0).start()
        idx_dma(0, 0).wait()
        gather_start(0)
        idx_dma(1, 1).start()

        def step(g, b, first, last, last2):
            nb = 1 - b
            gather_wait(b)
            if not last:
                if not first:
                    scatter_dma(0, nb).wait()
                idx_dma(0, nb).wait()
                gather_start(nb)
                if not last2:
                    idx_dma(g + 2, b).start()
            add_pos(b)
            scatter_dma(g, b).start()

        step(0, 0, True, False, False)

        def loop_body(g2, _):
            g = 2 * g2
            step(g + 1, 1, False, False, False)
            step(g + 2, 0, False, False, False)
            return _
        lax.fori_loop(0, (nseq_w - 4) // 2, loop_body, 0)

        step(nseq_w - 3, 1, False, False, False)
        step(nseq_w - 2, 0, False, False, True)
        step(nseq_w - 1, 1, False, True, True)
        scatter_dma(0, 0).wait()
        scatter_dma(0, 1).wait()

    return kern


_kern = _make_kernel()


def kernel(input_ids, token_table, position_table):
    rows = _BATCH * _SEQ
    ids = jnp.reshape(input_ids.astype(jnp.int32), (rows // _HALF, _HALF))
    out = _kern(ids, token_table, position_table)
    return jnp.reshape(out, (_BATCH, _SEQ, _EMBED))
